# baseline (speedup 1.0000x reference)
import jax
import jax.numpy as jnp
from jax import lax
from jax.experimental import pallas as pl
from jax.experimental.pallas import tpu as pltpu
from jax.experimental.pallas import tpu_sc as plsc

B, L, V = 1024, 50, 1000
R = B * L
NC, NS = 2, 16
NW = NC * NS
ROWS_PER_W = R // NW
CH = 32
CHW = CH * V
NCH = ROWS_PER_W // CH

_mesh = plsc.VectorSubcoreMesh(
    core_axis_name="c", subcore_axis_name="s", num_cores=NC, num_subcores=NS
)


def _body(x_hbm, out_hbm, idx_v, buf, sem):
    wid = lax.axis_index("s") * NC + lax.axis_index("c")
    base_row = wid * ROWS_PER_W

    pltpu.sync_copy(x_hbm.at[pl.ds(base_row, ROWS_PER_W)], idx_v)

    zeros16 = jnp.zeros((16,), jnp.float32)
    ones16 = jnp.ones((16,), jnp.float32)
    iota16 = lax.iota(jnp.int32, 16)

    def zero_body(k, carry):
        buf[pl.ds(k * 16, 16)] = zeros16
        return carry

    lax.fori_loop(0, CHW // 16, zero_body, 0)

    def scatter_chunk(c, val16):
        for j in range(CH // 16):
            rows = iota16 + (j * 16)
            xv = idx_v[pl.ds(c * CH + j * 16, 16)]
            plsc.store_scatter(buf, [rows * V + xv], val16)

    def chunk_body(c, carry):
        scatter_chunk(c, ones16)
        off = (base_row + c * CH) * V
        pltpu.sync_copy(buf, out_hbm.at[pl.ds(off, CHW)])
        scatter_chunk(c, zeros16)
        return carry

    lax.fori_loop(0, NCH, chunk_body, 0)


_onehot = pl.kernel(
    _body,
    out_type=jax.ShapeDtypeStruct((R * V,), jnp.float32),
    mesh=_mesh,
    scratch_types=[
        pltpu.VMEM((ROWS_PER_W,), jnp.int32),
        pltpu.VMEM((CHW,), jnp.float32),
        pltpu.SemaphoreType.DMA,
    ],
    compiler_params=pltpu.CompilerParams(needs_layout_passes=False),
)


@jax.jit
def kernel(x):
    flat = _onehot(x.astype(jnp.int32).reshape(R))
    return flat.reshape(B, L, V)
